# Initial kernel scaffold; baseline (speedup 1.0000x reference)
import jax
import jax.numpy as jnp
from jax.experimental import pallas as pl


def kernel(token_ids, embed_table, proj_w, scale):
    raise NotImplementedError("write your pallas kernel here")


# speedup vs baseline: 1.5838x; 1.5838x over previous
import functools

import jax
import jax.numpy as jnp
import numpy as np
from jax import lax
from jax.experimental import pallas as pl
from jax.experimental.pallas import tpu as pltpu
from jax.experimental.pallas import tpu_sc as plsc

_B_VOCAB = 100000
_P1 = 36313
_P2 = 27191
_D = 128
_MODEL_DIM = 2048

_NC = 2
_NS = 16
_NW = _NC * _NS
_LANES = 16
_IDX_GRP = 128


def _sc_hash_gather(cur, prev, table, n_tokens, seq_len):
    chunk = n_tokens // _NW
    ngrp = chunk // _IDX_GRP
    mesh = plsc.VectorSubcoreMesh(core_axis_name="c", subcore_axis_name="s")

    @functools.partial(
        pl.kernel,
        mesh=mesh,
        out_type=jax.ShapeDtypeStruct((n_tokens, _D), jnp.float32),
        scratch_types=[
            pltpu.VMEM((chunk,), jnp.int32),
            pltpu.VMEM((chunk,), jnp.int32),
            pltpu.VMEM((ngrp, _IDX_GRP), jnp.int32),
            pltpu.VMEM((chunk, _D), jnp.float32),
            pltpu.SemaphoreType.DMA,
        ],
    )
    def k(cur_hbm, prev_hbm, table_hbm, out_hbm, cur_v, prev_v, idx_v, rows_v, sem):
        wid = lax.axis_index("s") * _NC + lax.axis_index("c")
        base = wid * chunk
        pltpu.sync_copy(cur_hbm.at[pl.ds(base, chunk)], cur_v)
        pltpu.sync_copy(prev_hbm.at[pl.ds(base, chunk)], prev_v)
        lane = lax.iota(jnp.int32, _LANES)
        for i in range(chunk // _LANES):
            t = cur_v[pl.ds(i * _LANES, _LANES)].astype(jnp.uint32)
            p = prev_v[pl.ds(i * _LANES, _LANES)].astype(jnp.uint32)
            h = ((t * _P1) % _B_VOCAB + (p * _P2) % _B_VOCAB) % _B_VOCAB
            pos = base + i * _LANES + lane
            h = jnp.where((pos & (seq_len - 1)) == 0, jnp.uint32(0), h)
            g, r = divmod(i * _LANES, _IDX_GRP)
            idx_v[g, pl.ds(r, _LANES)] = h.astype(jnp.int32)
        for j in range(ngrp):
            pltpu.async_copy(
                table_hbm.at[idx_v.at[jnp.int32(j)]],
                rows_v.at[pl.ds(j * _IDX_GRP, _IDX_GRP)],
                sem,
            ).wait()
        pltpu.sync_copy(rows_v, out_hbm.at[pl.ds(base, chunk)])

    return k(cur, prev, table)


_ZERO = np.int32(0)


def _mm_body(s_ref, x_ref, w_ref, o_ref):
    x = x_ref[...] * s_ref[0]
    o_ref[...] = lax.dot_general(
        x, w_ref[...],
        dimension_numbers=(((1,), (1,)), ((), ())),
        preferred_element_type=jnp.float32,
    )


def _tc_project(rows, proj_w, scale, n_tokens):
    blk = 1024
    grid = n_tokens // blk
    return pl.pallas_call(
        _mm_body,
        grid=(grid,),
        in_specs=[
            pl.BlockSpec((1,), lambda i: (_ZERO,), memory_space=pltpu.SMEM),
            pl.BlockSpec((blk, _D), lambda i: (i, _ZERO)),
            pl.BlockSpec((_MODEL_DIM, _D), lambda i: (_ZERO, _ZERO)),
        ],
        out_specs=pl.BlockSpec((blk, _MODEL_DIM), lambda i: (i, _ZERO)),
        out_shape=jax.ShapeDtypeStruct((n_tokens, _MODEL_DIM), jnp.float32),
    )(scale.reshape((1,)).astype(jnp.float32), rows, proj_w)


def kernel(token_ids, embed_table, proj_w, scale):
    b, s = token_ids.shape
    n = b * s
    flat = token_ids.reshape((n,)).astype(jnp.int32)
    prev = jnp.concatenate([jnp.zeros((1,), jnp.int32), flat[:-1]])
    rows = _sc_hash_gather(flat, prev, embed_table, n, s)
    out = _tc_project(rows, proj_w, scale, n)
    return out.reshape((b, s, _MODEL_DIM))
